# baseline (speedup 1.0000x reference)
import functools

import jax
import jax.numpy as jnp
from jax import lax
from jax.experimental import pallas as pl
from jax.experimental.pallas import tpu as pltpu
from jax.experimental.pallas import tpu_sc as plsc

EMBEDDING_DIM = 64
PAD_DIM = 128
BATCH = 16384
HIST = 50
HIST_PAD = 56
NUM_EMB = 1000000
NUM_CORES = 2
NUM_SUBCORES = 16
NUM_WORKERS = NUM_CORES * NUM_SUBCORES
BLOCK = BATCH // NUM_WORKERS

VB = 384
V16 = VB // 16
NFULL = NUM_EMB // VB
BASE_TRIPS = NFULL // NUM_WORKERS
EXTRA = NFULL % NUM_WORKERS
TAIL_V0 = NFULL * VB
TAIL_W = NUM_EMB - TAIL_V0

_mesh = plsc.VectorSubcoreMesh(core_axis_name="c", subcore_axis_name="s")


@functools.partial(
    pl.kernel,
    mesh=_mesh,
    out_type=jax.ShapeDtypeStruct((NUM_EMB * EMBEDDING_DIM,), jnp.float32),
    scratch_types=[
        pltpu.VMEM((EMBEDDING_DIM, VB), jnp.float32),
        pltpu.VMEM((EMBEDDING_DIM, VB), jnp.float32),
        pltpu.VMEM((VB * EMBEDDING_DIM,), jnp.float32),
        pltpu.VMEM((VB * EMBEDDING_DIM,), jnp.float32),
        pltpu.SemaphoreType.DMA,
        pltpu.SemaphoreType.DMA,
        pltpu.SemaphoreType.DMA,
        pltpu.SemaphoreType.DMA,
    ],
    compiler_params=pltpu.CompilerParams(needs_layout_passes=False),
)
def _detile(src_hbm, tail_hbm, out_hbm, s0, s1, d0, d1, l0, l1, t0, t1):
    wid = lax.axis_index("s") * NUM_CORES + lax.axis_index("c")
    src = (s0, s1)
    dst = (d0, d1)
    lsem = (l0, l1)
    ssem = (t0, t1)
    trips = BASE_TRIPS + jnp.where(wid < EXTRA, 1, 0)
    start = BASE_TRIPS * wid + jnp.minimum(wid, EXTRA)
    iota64 = lax.iota(jnp.int32, 16) * EMBEDDING_DIM

    def start_load(i, b):
        pltpu.async_copy(src_hbm.at[:, pl.ds((start + i) * VB, VB)], src[b], lsem[b])

    def wait_load(b):
        pltpu.make_async_copy(src_hbm.at[:, pl.ds(0, VB)], src[b], lsem[b]).wait()

    def start_store(i, b):
        pltpu.async_copy(
            dst[b], out_hbm.at[pl.ds((start + i) * (VB * EMBEDDING_DIM), VB * EMBEDDING_DIM)], ssem[b]
        )

    def wait_store(b):
        pltpu.make_async_copy(
            dst[b], out_hbm.at[pl.ds(0, VB * EMBEDDING_DIM)], ssem[b]
        ).wait()

    def transpose(b, ngroups):
        def tb(v16, carry):
            base = v16 * (16 * EMBEDDING_DIM)
            for d in range(EMBEDDING_DIM):
                val = src[b][d, pl.ds(v16 * 16, 16)]
                plsc.store_scatter(dst[b], [iota64 + (base + d)], val)
            return carry

        lax.fori_loop(0, ngroups, tb, 0, unroll=False)

    start_load(0, 0)

    @pl.when(trips > 1)
    def _():
        start_load(1, 1)

    def body(i, carry):
        def do(b):
            wait_load(b)

            @pl.when(i >= 2)
            def _():
                wait_store(b)

            transpose(b, V16)
            start_store(i, b)

            @pl.when(i + 2 < trips)
            def _():
                start_load(i + 2, b)

        @pl.when(i % 2 == 0)
        def _():
            do(0)

        @pl.when(i % 2 == 1)
        def _():
            do(1)

        return carry

    lax.fori_loop(0, trips, body, 0, unroll=False)
    wait_store(0)

    @pl.when(trips > 1)
    def _():
        wait_store(1)

    @pl.when(wid == NUM_WORKERS - 1)
    def _():
        n = TAIL_W * EMBEDDING_DIM
        pltpu.sync_copy(tail_hbm, d0.at[pl.ds(0, n)])
        pltpu.sync_copy(
            d0.at[pl.ds(0, n)],
            out_hbm.at[pl.ds(TAIL_V0 * EMBEDDING_DIM, n)],
        )


@functools.partial(
    pl.kernel,
    mesh=_mesh,
    out_type=jax.ShapeDtypeStruct((BATCH, HIST_PAD, PAD_DIM), jnp.float32),
    scratch_types=[
        pltpu.VMEM((HIST, BLOCK), jnp.int32),
        pltpu.VMEM((BLOCK, EMBEDDING_DIM), jnp.float32),
        pltpu.VMEM((BLOCK, EMBEDDING_DIM), jnp.float32),
        pltpu.SemaphoreType.DMA,
        pltpu.SemaphoreType.DMA,
        pltpu.SemaphoreType.DMA,
        pltpu.SemaphoreType.DMA,
    ],
    compiler_params=pltpu.CompilerParams(use_tc_tiling_on_sc=False),
)
def _gather(idx_hbm, table_hbm, out_hbm, idx_v, rows0, rows1, g0, g1, s0, s1):
    wid = lax.axis_index("s") * NUM_CORES + lax.axis_index("c")
    b0 = wid * BLOCK
    rows = (rows0, rows1)
    gsem = (g0, g1)
    ssem = (s0, s1)

    pltpu.sync_copy(idx_hbm.at[:, pl.ds(b0, BLOCK)], idx_v)

    def start_gather(h, b):
        pltpu.async_copy(table_hbm.at[idx_v.at[h]], rows[b], gsem[b])

    def wait_gather(b):
        pltpu.make_async_copy(table_hbm.at[idx_v.at[0]], rows[b], gsem[b]).wait()

    def start_store(h, b):
        pltpu.async_copy(
            rows[b], out_hbm.at[pl.ds(b0, BLOCK), h, pl.ds(0, EMBEDDING_DIM)], ssem[b]
        )

    def wait_store(b):
        pltpu.make_async_copy(
            rows[b], out_hbm.at[pl.ds(b0, BLOCK), 0, pl.ds(0, EMBEDDING_DIM)], ssem[b]
        ).wait()

    start_gather(0, 0)
    start_gather(1, 1)

    def body(k, carry):
        wait_gather(0)
        start_store(k, 0)
        wait_store(0)
        start_gather(k + 2, 0)
        wait_gather(1)
        start_store(k + 1, 1)
        wait_store(1)
        start_gather(k + 3, 1)
        return carry

    lax.fori_loop(0, (HIST - 2) // 2, lambda i, c: body(2 * i, c), 0, unroll=False)

    wait_gather(0)
    start_store(HIST - 2, 0)
    wait_gather(1)
    start_store(HIST - 1, 1)
    wait_store(0)
    wait_store(1)


def kernel(token_ids, embeddings):
    tail_flat = embeddings[TAIL_V0:].reshape(TAIL_W * EMBEDDING_DIM)
    table_flat = _detile(embeddings.T, tail_flat)
    table = table_flat.reshape(NUM_EMB, EMBEDDING_DIM)
    out = _gather(token_ids.T, table)
    return out[:, :HIST, :EMBEDDING_DIM]
